# baseline (speedup 1.0000x reference)
import functools

import jax
import jax.numpy as jnp
from jax import lax
from jax.experimental import pallas as pl
from jax.experimental.pallas import tpu as pltpu
from jax.experimental.pallas import tpu_sc as plsc

N = 10000
D = 128
E = 320000
NC = 2
NS = 16
NW = NC * NS
CH = 128
NCHUNK_W = 80
GW = 20
NGRP = NCHUNK_W // GW
EPW = CH * NCHUNK_W
EP = EPW * NW
NCHUNK_TOTAL = EP // CH
ROWS_PER_TILE = 624
ROWS_TAIL = N - NS * ROWS_PER_TILE


def _leaky(v):
    return jnp.where(v >= 0, v, 0.01 * v)


def _bn_body(x_ref, g_ref, b_ref, h_ref):
    x = x_ref[...]
    mean = jnp.mean(x, axis=0, keepdims=True)
    var = jnp.mean((x - mean) ** 2, axis=0, keepdims=True)
    h = (x - mean) / jnp.sqrt(var + 1e-5) * g_ref[...] + b_ref[...]
    h_ref[...] = _leaky(h)


def _bn(x, gamma, beta):
    return pl.pallas_call(
        _bn_body,
        out_shape=jax.ShapeDtypeStruct((N, D), jnp.float32),
    )(x, gamma.reshape(1, D), beta.reshape(1, D))


_EB = 6400


def _edge_body(ea_ref, w1_ref, b1_ref, w2_ref, b2_ref, out_ref):
    a = jnp.dot(ea_ref[...], w1_ref[...], preferred_element_type=jnp.float32)
    a = _leaky(a + b1_ref[...])
    v = jnp.dot(a, w2_ref[...], preferred_element_type=jnp.float32) + b2_ref[...]
    out_ref[...] = jnp.where(v > 0, v, jnp.exp(v) - 1.0)


def _edge_w(edge_attr, et_w1, et_b1, et_w2, et_b2):
    grid = E // _EB
    return pl.pallas_call(
        _edge_body,
        grid=(grid,),
        in_specs=[
            pl.BlockSpec((_EB, 16), lambda i: (i, 0)),
            pl.BlockSpec((16, 8), lambda i: (0, 0)),
            pl.BlockSpec((1, 8), lambda i: (0, 0)),
            pl.BlockSpec((8, 1), lambda i: (0, 0)),
            pl.BlockSpec((1, 1), lambda i: (0, 0)),
        ],
        out_specs=pl.BlockSpec((_EB, 1), lambda i: (i, 0)),
        out_shape=jax.ShapeDtypeStruct((E, 1), jnp.float32),
    )(edge_attr, et_w1, et_b1.reshape(1, 8), et_w2, et_b2.reshape(1, 1))


_MESH = plsc.VectorSubcoreMesh(
    core_axis_name="c", subcore_axis_name="s", num_cores=NC, num_subcores=NS)


@functools.partial(
    pl.kernel,
    out_type=jax.ShapeDtypeStruct((NC, N, D), jnp.float32),
    mesh=_MESH,
    scratch_types=[
        pltpu.VMEM((GW, 1, CH), jnp.int32),
        pltpu.VMEM((GW, 1, CH), jnp.int32),
        pltpu.VMEM((GW, 1, CH + 16), jnp.float32),
        pltpu.VMEM((2, CH, D), jnp.float32),
        pltpu.VMEM_SHARED((N, D), jnp.float32),
        pltpu.SemaphoreType.DMA,
        pltpu.SemaphoreType.DMA,
        pltpu.SemaphoreType.DMA,
    ],
)
def _sc_aggregate(h_hbm, src_hbm, dst_hbm, w_hbm, zero_hbm, out_hbm,
                  src_v, dst_v, w_v, rows_v, acc, gsem0, gsem1, ssem):
    c = lax.axis_index("c")
    s = lax.axis_index("s")
    wid = s * NC + c

    pltpu.sync_copy(zero_hbm,
                    acc.at[pl.ds(s * ROWS_PER_TILE, ROWS_PER_TILE)])

    @pl.when(s == 0)
    def _zero_tail():
        pltpu.sync_copy(zero_hbm.at[pl.ds(0, ROWS_TAIL)],
                        acc.at[pl.ds(NS * ROWS_PER_TILE, ROWS_TAIL)])

    base = wid * NCHUNK_W
    plsc.subcore_barrier()

    def group(grp, carry):
        gbase = base + grp * GW
        pltpu.sync_copy(src_hbm.at[pl.ds(gbase, GW)], src_v)
        pltpu.sync_copy(dst_hbm.at[pl.ds(gbase, GW)], dst_v)
        pltpu.sync_copy(w_hbm.at[pl.ds(gbase, GW)], w_v)

        pltpu.async_copy(h_hbm.at[src_v.at[0, 0]], rows_v.at[0], gsem0)

        def emit_window(g, buf, sem_this, sem_other, first):
            @pl.when(jnp.logical_not(first))
            def _wait_scatter():
                pltpu.make_async_copy(rows_v.at[1 - buf],
                                      acc.at[dst_v.at[g, 0]], ssem).wait()

            @pl.when(g + 1 < GW)
            def _next_gather():
                pltpu.async_copy(h_hbm.at[src_v.at[g + 1, 0]],
                                 rows_v.at[1 - buf], sem_other)

            pltpu.make_async_copy(h_hbm.at[src_v.at[g, 0]], rows_v.at[buf],
                                  sem_this).wait()

            def _scale(e):
                wval = w_v[g, 0, pl.ds(e, 16)][0]
                for j in range(D // 16):
                    sl = pl.ds(j * 16, 16)
                    rows_v[buf, e, sl] = rows_v[buf, e, sl] * wval

            plsc.parallel_loop(0, CH, 1, unroll=4)(_scale)

            pltpu.async_copy(rows_v.at[buf], acc.at[dst_v.at[g, 0]], ssem,
                             add=True)

        def pair(q, carry2):
            g0 = 2 * q
            emit_window(g0, 0, gsem0, gsem1, first=(q == 0))
            emit_window(g0 + 1, 1, gsem1, gsem0, first=jnp.bool_(False))
            return carry2

        lax.fori_loop(0, GW // 2, pair, 0, unroll=False)
        pltpu.make_async_copy(rows_v.at[1], acc.at[dst_v.at[GW - 1, 0]],
                              ssem).wait()
        return carry

    lax.fori_loop(0, NGRP, group, 0, unroll=False)
    plsc.subcore_barrier()

    pltpu.sync_copy(acc.at[pl.ds(s * ROWS_PER_TILE, ROWS_PER_TILE)],
                    out_hbm.at[c, pl.ds(s * ROWS_PER_TILE, ROWS_PER_TILE)])

    @pl.when(s == 0)
    def _out_tail():
        pltpu.sync_copy(acc.at[pl.ds(NS * ROWS_PER_TILE, ROWS_TAIL)],
                        out_hbm.at[c, pl.ds(NS * ROWS_PER_TILE, ROWS_TAIL)])


_RB = 1000


def _mlp_body(p_ref, h_ref, x_ref, w1_ref, b1_ref, w2_ref, b2_ref, out_ref):
    a = p_ref[0] + p_ref[1] + h_ref[...]
    t = jnp.dot(a, w1_ref[...], preferred_element_type=jnp.float32) + b1_ref[...]
    t = _leaky(t)
    out_ref[...] = (jnp.dot(t, w2_ref[...], preferred_element_type=jnp.float32)
                    + b2_ref[...] + x_ref[...])


def _mlp(partials, h, x, nn_w1, nn_b1, nn_w2, nn_b2):
    grid = N // _RB
    return pl.pallas_call(
        _mlp_body,
        grid=(grid,),
        in_specs=[
            pl.BlockSpec((NC, _RB, D), lambda i: (0, i, 0)),
            pl.BlockSpec((_RB, D), lambda i: (i, 0)),
            pl.BlockSpec((_RB, D), lambda i: (i, 0)),
            pl.BlockSpec((D, D), lambda i: (0, 0)),
            pl.BlockSpec((1, D), lambda i: (0, 0)),
            pl.BlockSpec((D, D), lambda i: (0, 0)),
            pl.BlockSpec((1, D), lambda i: (0, 0)),
        ],
        out_specs=pl.BlockSpec((_RB, D), lambda i: (i, 0)),
        out_shape=jax.ShapeDtypeStruct((N, D), jnp.float32),
    )(partials, h, x, nn_w1, nn_b1.reshape(1, D), nn_w2, nn_b2.reshape(1, D))


def kernel(x, edge_index, edge_attr, bn_gamma, bn_beta,
           et_w1, et_b1, et_w2, et_b2,
           nn_w1, nn_b1, nn_w2, nn_b2):
    h = _bn(x, bn_gamma, bn_beta)
    w = _edge_w(edge_attr, et_w1, et_b1, et_w2, et_b2)[:, 0]

    src = edge_index[0].astype(jnp.int32)
    dst = edge_index[1].astype(jnp.int32)
    pad = EP - E
    pad_idx = jnp.arange(pad, dtype=jnp.int32) % N
    src_p = jnp.concatenate([src, pad_idx]).reshape(NCHUNK_TOTAL, 1, CH)
    dst_p = jnp.concatenate([dst, pad_idx]).reshape(NCHUNK_TOTAL, 1, CH)
    w_p = jnp.concatenate([w, jnp.zeros((pad,), jnp.float32)]).reshape(
        NCHUNK_TOTAL, 1, CH)
    w_p = jnp.pad(w_p, ((0, 0), (0, 0), (0, 16)))
    zeros = jnp.zeros((ROWS_PER_TILE, D), jnp.float32)

    partials = _sc_aggregate(h, src_p, dst_p, w_p, zeros)
    return _mlp(partials, h, x, nn_w1, nn_b1, nn_w2, nn_b2)


# speedup vs baseline: 1.7976x; 1.7976x over previous
import functools

import jax
import jax.numpy as jnp
from jax import lax
from jax.experimental import pallas as pl
from jax.experimental.pallas import tpu as pltpu
from jax.experimental.pallas import tpu_sc as plsc

N = 10000
D = 128
E = 320000
NC = 2
NS = 16
NW = NC * NS
CH = 128
NCHUNK_W = 80
GW = 20
NGRP = NCHUNK_W // GW
EPW = CH * NCHUNK_W
EP = EPW * NW
NCHUNK_TOTAL = EP // CH
ROWS_PER_TILE = 624
ROWS_TAIL = N - NS * ROWS_PER_TILE


def _leaky(v):
    return jnp.where(v >= 0, v, 0.01 * v)


def _bn_body(x_ref, g_ref, b_ref, h_ref):
    x = x_ref[...]
    mean = jnp.mean(x, axis=0, keepdims=True)
    var = jnp.mean((x - mean) ** 2, axis=0, keepdims=True)
    h = (x - mean) / jnp.sqrt(var + 1e-5) * g_ref[...] + b_ref[...]
    h_ref[...] = _leaky(h)


def _bn(x, gamma, beta):
    return pl.pallas_call(
        _bn_body,
        out_shape=jax.ShapeDtypeStruct((N, D), jnp.float32),
    )(x, gamma.reshape(1, D), beta.reshape(1, D))


_EB = 6400


def _edge_body(eat_ref, w1t_ref, b1_ref, w2t_ref, b2_ref, out_ref):
    a = jnp.dot(w1t_ref[...], eat_ref[...], preferred_element_type=jnp.float32)
    a = _leaky(a + b1_ref[...])
    v = jnp.dot(w2t_ref[...], a, preferred_element_type=jnp.float32) + b2_ref[...]
    out_ref[...] = jnp.where(v > 0, v, jnp.exp(v) - 1.0)


def _edge_w(edge_attr, et_w1, et_b1, et_w2, et_b2):
    eat = edge_attr.T
    grid = E // _EB
    return pl.pallas_call(
        _edge_body,
        grid=(grid,),
        in_specs=[
            pl.BlockSpec((16, _EB), lambda i: (0, i)),
            pl.BlockSpec((8, 16), lambda i: (0, 0)),
            pl.BlockSpec((8, 1), lambda i: (0, 0)),
            pl.BlockSpec((1, 8), lambda i: (0, 0)),
            pl.BlockSpec((1, 1), lambda i: (0, 0)),
        ],
        out_specs=pl.BlockSpec((1, _EB), lambda i: (0, i)),
        out_shape=jax.ShapeDtypeStruct((1, E), jnp.float32),
    )(eat, et_w1.T, et_b1.reshape(8, 1), et_w2.T, et_b2.reshape(1, 1))


_MESH = plsc.VectorSubcoreMesh(
    core_axis_name="c", subcore_axis_name="s", num_cores=NC, num_subcores=NS)


@functools.partial(
    pl.kernel,
    out_type=jax.ShapeDtypeStruct((NC, N, D), jnp.float32),
    mesh=_MESH,
    scratch_types=[
        pltpu.VMEM((GW, 1, CH), jnp.int32),
        pltpu.VMEM((GW, 1, CH), jnp.int32),
        pltpu.VMEM((GW, 1, CH + 16), jnp.float32),
        pltpu.VMEM((2, CH, D), jnp.float32),
        pltpu.VMEM_SHARED((N, D), jnp.float32),
        pltpu.SemaphoreType.DMA,
        pltpu.SemaphoreType.DMA,
        pltpu.SemaphoreType.DMA,
    ],
)
def _sc_aggregate(h_hbm, src_hbm, dst_hbm, w_hbm, zero_hbm, out_hbm,
                  src_v, dst_v, w_v, rows_v, acc, gsem0, gsem1, ssem):
    c = lax.axis_index("c")
    s = lax.axis_index("s")
    wid = s * NC + c

    pltpu.sync_copy(zero_hbm,
                    acc.at[pl.ds(s * ROWS_PER_TILE, ROWS_PER_TILE)])

    @pl.when(s == 0)
    def _zero_tail():
        pltpu.sync_copy(zero_hbm.at[pl.ds(0, ROWS_TAIL)],
                        acc.at[pl.ds(NS * ROWS_PER_TILE, ROWS_TAIL)])

    base = wid * NCHUNK_W
    plsc.subcore_barrier()

    def group(grp, carry):
        gbase = base + grp * GW
        pltpu.sync_copy(src_hbm.at[pl.ds(gbase, GW)], src_v)
        pltpu.sync_copy(dst_hbm.at[pl.ds(gbase, GW)], dst_v)
        pltpu.sync_copy(w_hbm.at[pl.ds(gbase, GW)], w_v)

        pltpu.async_copy(h_hbm.at[src_v.at[0, 0]], rows_v.at[0], gsem0)

        def emit_window(g, buf, sem_this, sem_other, first):
            @pl.when(jnp.logical_not(first))
            def _wait_scatter():
                pltpu.make_async_copy(rows_v.at[1 - buf],
                                      acc.at[dst_v.at[g, 0]], ssem).wait()

            @pl.when(g + 1 < GW)
            def _next_gather():
                pltpu.async_copy(h_hbm.at[src_v.at[g + 1, 0]],
                                 rows_v.at[1 - buf], sem_other)

            pltpu.make_async_copy(h_hbm.at[src_v.at[g, 0]], rows_v.at[buf],
                                  sem_this).wait()

            def _scale(e):
                wval = w_v[g, 0, pl.ds(e, 16)][0]
                for j in range(D // 16):
                    sl = pl.ds(j * 16, 16)
                    rows_v[buf, e, sl] = rows_v[buf, e, sl] * wval

            plsc.parallel_loop(0, CH, 1, unroll=4)(_scale)

            pltpu.async_copy(rows_v.at[buf], acc.at[dst_v.at[g, 0]], ssem,
                             add=True)

        def pair(q, carry2):
            g0 = 2 * q
            emit_window(g0, 0, gsem0, gsem1, first=(q == 0))
            emit_window(g0 + 1, 1, gsem1, gsem0, first=jnp.bool_(False))
            return carry2

        lax.fori_loop(0, GW // 2, pair, 0, unroll=False)
        pltpu.make_async_copy(rows_v.at[1], acc.at[dst_v.at[GW - 1, 0]],
                              ssem).wait()
        return carry

    lax.fori_loop(0, NGRP, group, 0, unroll=False)
    plsc.subcore_barrier()

    pltpu.sync_copy(acc.at[pl.ds(s * ROWS_PER_TILE, ROWS_PER_TILE)],
                    out_hbm.at[c, pl.ds(s * ROWS_PER_TILE, ROWS_PER_TILE)])

    @pl.when(s == 0)
    def _out_tail():
        pltpu.sync_copy(acc.at[pl.ds(NS * ROWS_PER_TILE, ROWS_TAIL)],
                        out_hbm.at[c, pl.ds(NS * ROWS_PER_TILE, ROWS_TAIL)])


_RB = 1000


def _mlp_body(p_ref, h_ref, x_ref, w1_ref, b1_ref, w2_ref, b2_ref, out_ref):
    a = p_ref[0] + p_ref[1] + h_ref[...]
    t = jnp.dot(a, w1_ref[...], preferred_element_type=jnp.float32) + b1_ref[...]
    t = _leaky(t)
    out_ref[...] = (jnp.dot(t, w2_ref[...], preferred_element_type=jnp.float32)
                    + b2_ref[...] + x_ref[...])


def _mlp(partials, h, x, nn_w1, nn_b1, nn_w2, nn_b2):
    grid = N // _RB
    return pl.pallas_call(
        _mlp_body,
        grid=(grid,),
        in_specs=[
            pl.BlockSpec((NC, _RB, D), lambda i: (0, i, 0)),
            pl.BlockSpec((_RB, D), lambda i: (i, 0)),
            pl.BlockSpec((_RB, D), lambda i: (i, 0)),
            pl.BlockSpec((D, D), lambda i: (0, 0)),
            pl.BlockSpec((1, D), lambda i: (0, 0)),
            pl.BlockSpec((D, D), lambda i: (0, 0)),
            pl.BlockSpec((1, D), lambda i: (0, 0)),
        ],
        out_specs=pl.BlockSpec((_RB, D), lambda i: (i, 0)),
        out_shape=jax.ShapeDtypeStruct((N, D), jnp.float32),
    )(partials, h, x, nn_w1, nn_b1.reshape(1, D), nn_w2, nn_b2.reshape(1, D))


def kernel(x, edge_index, edge_attr, bn_gamma, bn_beta,
           et_w1, et_b1, et_w2, et_b2,
           nn_w1, nn_b1, nn_w2, nn_b2):
    h = _bn(x, bn_gamma, bn_beta)
    w = _edge_w(edge_attr, et_w1, et_b1, et_w2, et_b2)[0]

    src = edge_index[0].astype(jnp.int32)
    dst = edge_index[1].astype(jnp.int32)
    pad = EP - E
    pad_idx = jnp.arange(pad, dtype=jnp.int32) % N
    src_p = jnp.concatenate([src, pad_idx]).reshape(NCHUNK_TOTAL, 1, CH)
    dst_p = jnp.concatenate([dst, pad_idx]).reshape(NCHUNK_TOTAL, 1, CH)
    w_p = jnp.concatenate([w, jnp.zeros((pad,), jnp.float32)]).reshape(
        NCHUNK_TOTAL, 1, CH)
    w_p = jnp.pad(w_p, ((0, 0), (0, 0), (0, 16)))
    zeros = jnp.zeros((ROWS_PER_TILE, D), jnp.float32)

    partials = _sc_aggregate(h, src_p, dst_p, w_p, zeros)
    return _mlp(partials, h, x, nn_w1, nn_b1, nn_w2, nn_b2)
